# baseline (speedup 1.0000x reference)
import functools
import jax
import jax.numpy as jnp
from jax import lax
from jax.experimental import pallas as pl
from jax.experimental.pallas import tpu as pltpu
from jax.experimental.pallas import tpu_sc as plsc

N = 10000
E = 320000
D = 128
T = 4
L = 16
NW = 32
CH = 128
DV = D // L
EPW1 = 6144
EPW2 = E // NW - EPW1


def _make_kernel(e0, epw):
  nfull = epw // CH
  tail = epw - nfull * CH
  npair = nfull // 2
  assert nfull % 2 == 0 and tail % T == 0
  mesh = plsc.VectorSubcoreMesh(core_axis_name="c", subcore_axis_name="s")

  @functools.partial(
      pl.kernel,
      mesh=mesh,
      out_type=jax.ShapeDtypeStruct((epw * NW, T), jnp.float32),
      compiler_params=pltpu.CompilerParams(needs_layout_passes=False),
      scratch_types=[
          pltpu.VMEM((epw,), jnp.int32),
          pltpu.VMEM((epw,), jnp.int32),
          pltpu.VMEM((CH, D), jnp.int32),
          pltpu.VMEM((CH, D), jnp.int32),
          pltpu.VMEM((CH, D), jnp.int32),
          pltpu.VMEM((CH, D), jnp.int32),
          pltpu.VMEM((CH, T), jnp.float32),
          pltpu.VMEM((CH, T), jnp.float32),
          pltpu.VMEM((T, DV // 2, 2, L), jnp.float32),
          pltpu.VMEM((L,), jnp.float32),
          pltpu.VMEM((CH // T // 2 * L * L,), jnp.float32),
          pltpu.VMEM((L * L,), jnp.int32),
          pltpu.SemaphoreType.DMA,
          pltpu.SemaphoreType.DMA,
          pltpu.SemaphoreType.DMA,
          pltpu.SemaphoreType.DMA,
          pltpu.SemaphoreType.DMA,
          pltpu.SemaphoreType.DMA,
      ],
  )
  def k(table_hbm, src_hbm, dst_hbm, wr_hbm, binit_hbm, out_hbm2,
        sidx, didx, srA, drA, srB, drB, outA, outB, wr_v, b_v, pbufs,
        idxb, gsA, gdA, gsB, gdB, oA, oB):
    wid = lax.axis_index("s") * 2 + lax.axis_index("c")
    base = wid * epw

    pltpu.sync_copy(wr_hbm, wr_v)
    pltpu.sync_copy(binit_hbm, b_v)
    pltpu.sync_copy(src_hbm.at[pl.ds(e0 + base, epw)], sidx)
    pltpu.sync_copy(dst_hbm.at[pl.ds(e0 + base, epw)], didx)

    wvec = [
        [wr_v[t, v4, hf, :] for v4 in range(DV // 2) for hf in range(2)]
        for t in range(T)
    ]
    btile = b_v[:]
    lane16 = lax.iota(jnp.int32, L) * L
    for m in range(L):
      idxb[pl.ds(m * L, L)] = lane16 + m
    lane = lax.iota(jnp.int32, L)
    row4 = lax.shift_right_logical(lane, 2)
    col4 = lane - row4 * T

    def issue(k_chunk, sr, dr, gs, gd):
      off = k_chunk * CH
      pltpu.async_copy(table_hbm.at[sidx.at[pl.ds(off, CH)]], sr, gs)
      pltpu.async_copy(table_hbm.at[didx.at[pl.ds(off, CH)]], dr, gd)

    def wait_gathers(sr, dr, gs, gd):
      pltpu.make_async_copy(table_hbm.at[sidx.at[pl.ds(0, CH)]], sr, gs).wait()
      pltpu.make_async_copy(table_hbm.at[didx.at[pl.ds(0, CH)]], dr, gd).wait()

    def compute(sr, dr, ob, ngrp):
      nhalf = max(ngrp // 2, 1)

      def half_loop(g0):
        @plsc.parallel_loop(g0, g0 + nhalf, unroll=1)
        def group(g):
          pb = pbufs.at[pl.ds((g - g0) * L * L, L * L)]
          for j in range(T):
            e = T * g + j
            h = []
            for v4 in range(DV // 2):
              sbf = plsc.bitcast(sr[e, L * v4:L * (v4 + 1)], jnp.bfloat16)
              dbf = plsc.bitcast(dr[e, L * v4:L * (v4 + 1)], jnp.bfloat16)
              h32 = jnp.maximum(sbf + dbf, 0.0)
              ha, hb = plsc.unpack(h32, format=plsc.PackFormat.INTERLEAVED)
              h.append(ha)
              h.append(hb)
            for t in range(T):
              ps = [h[v] * wvec[t][v] for v in range(DV)]
              while len(ps) > 1:
                ps = [ps[x] + ps[x + 1] for x in range(0, len(ps), 2)]
              m = j * T + t
              plsc.store_scatter(pb, [idxb[pl.ds(m * L, L)]], ps[0])
          ov = pb[0:L] + btile
          for r in range(1, L):
            ov = ov + pb[L * r:L * (r + 1)]
          plsc.store_scatter(ob, [row4 + g * T, col4], ov)

      half_loop(0)
      if ngrp > 1:
        half_loop(nhalf)

    def out_start(k_chunk, ob, sem):
      pltpu.async_copy(
          ob, out_hbm2.at[pl.ds(base + k_chunk * CH, CH)], sem)

    def out_wait(ob, sem):
      pltpu.make_async_copy(
          ob, out_hbm2.at[pl.ds(base, CH)], sem).wait()

    issue(0, srA, drA, gsA, gdA)
    issue(1, srB, drB, gsB, gdB)

    def pair_body(i, _):
      k0 = 2 * i
      wait_gathers(srA, drA, gsA, gdA)

      @pl.when(i > 0)
      def _wA():
        out_wait(outA, oA)

      compute(srA, drA, outA, CH // T)
      out_start(k0, outA, oA)

      @pl.when(i < npair - 1)
      def _iA():
        issue(k0 + 2, srA, drA, gsA, gdA)

      wait_gathers(srB, drB, gsB, gdB)

      @pl.when(i > 0)
      def _wB():
        out_wait(outB, oB)

      compute(srB, drB, outB, CH // T)
      out_start(k0 + 1, outB, oB)

      @pl.when(i < npair - 1)
      def _iB():
        issue(k0 + 3, srB, drB, gsB, gdB)

      return _

    lax.fori_loop(0, npair, pair_body, None)

    if tail:
      toff = nfull * CH
      pltpu.async_copy(
          table_hbm.at[sidx.at[pl.ds(toff, tail)]], srA.at[pl.ds(0, tail)],
          gsA)
      pltpu.async_copy(
          table_hbm.at[didx.at[pl.ds(toff, tail)]], drA.at[pl.ds(0, tail)],
          gdA)
      pltpu.make_async_copy(
          table_hbm.at[sidx.at[pl.ds(toff, tail)]], srA.at[pl.ds(0, tail)],
          gsA).wait()
      pltpu.make_async_copy(
          table_hbm.at[didx.at[pl.ds(toff, tail)]], drA.at[pl.ds(0, tail)],
          gdA).wait()
      out_wait(outA, oA)
      compute(srA, drA, outA, tail // T)
      out_wait(outB, oB)
      pltpu.sync_copy(
          outA.at[pl.ds(0, tail), :],
          out_hbm2.at[pl.ds(base + toff, tail)])
    else:
      out_wait(outA, oA)
      out_wait(outB, oB)

  return k


_kern1 = _make_kernel(0, EPW1)
_kern2 = _make_kernel(EPW1 * NW, EPW2)


def kernel(encoded_graph, edge_index, W, b):
  ei = edge_index.astype(jnp.int32)
  table_bf = jnp.pad(
      jax.lax.bitcast_convert_type(
          encoded_graph.astype(jnp.bfloat16).reshape(N, D // 2, 2), jnp.int32),
      ((0, 0), (0, D // 2)))
  wr = W.T.reshape(T, DV // 2, L, 2).transpose(0, 1, 3, 2)
  binit = jnp.tile(b, L // T)
  src = ei[0]
  dst = ei[1]
  o1 = _kern1(table_bf, src, dst, wr, binit)
  o2 = _kern2(table_bf, src, dst, wr, binit)
  return jnp.concatenate([o1, o2], axis=0)


# speedup vs baseline: 1.0673x; 1.0673x over previous
import functools
import jax
import jax.numpy as jnp
from jax import lax
from jax.experimental import pallas as pl
from jax.experimental.pallas import tpu as pltpu
from jax.experimental.pallas import tpu_sc as plsc

N = 10000
E = 320000
D = 128
T = 4
L = 16
NW = 32
CH = 128
DV = D // L
EPW1 = 6144
EPW2 = E // NW - EPW1


def _make_kernel(e0, epw):
  nfull = epw // CH
  tail = epw - nfull * CH
  npair = nfull // 2
  assert nfull % 2 == 0 and tail % T == 0
  mesh = plsc.VectorSubcoreMesh(core_axis_name="c", subcore_axis_name="s")

  @functools.partial(
      pl.kernel,
      mesh=mesh,
      out_type=jax.ShapeDtypeStruct((epw * NW, T), jnp.float32),
      compiler_params=pltpu.CompilerParams(needs_layout_passes=False),
      scratch_types=[
          pltpu.VMEM((epw,), jnp.int32),
          pltpu.VMEM((epw,), jnp.int32),
          pltpu.VMEM((CH, D), jnp.int32),
          pltpu.VMEM((CH, D), jnp.int32),
          pltpu.VMEM((CH, D), jnp.int32),
          pltpu.VMEM((CH, D), jnp.int32),
          pltpu.VMEM((CH, T), jnp.float32),
          pltpu.VMEM((CH, T), jnp.float32),
          pltpu.VMEM((T, DV // 2, 2, L), jnp.float32),
          pltpu.VMEM((L,), jnp.float32),
          pltpu.VMEM((CH // T // 2 * L * L,), jnp.float32),
          pltpu.VMEM((L * L,), jnp.int32),
          pltpu.SemaphoreType.DMA,
          pltpu.SemaphoreType.DMA,
          pltpu.SemaphoreType.DMA,
          pltpu.SemaphoreType.DMA,
          pltpu.SemaphoreType.DMA,
          pltpu.SemaphoreType.DMA,
      ],
  )
  def k(table_hbm, src_hbm, dst_hbm, wr_hbm, binit_hbm, out_hbm2,
        sidx, didx, srA, drA, srB, drB, outA, outB, wr_v, b_v, pbufs,
        idxb, gsA, gdA, gsB, gdB, oA, oB):
    wid = lax.axis_index("s") * 2 + lax.axis_index("c")
    base = wid * epw

    pltpu.sync_copy(wr_hbm, wr_v)
    pltpu.sync_copy(binit_hbm, b_v)
    pltpu.sync_copy(src_hbm.at[pl.ds(e0 + base, epw)], sidx)
    pltpu.sync_copy(dst_hbm.at[pl.ds(e0 + base, epw)], didx)

    wvec = [
        [wr_v[t, v4, hf, :] for v4 in range(DV // 2) for hf in range(2)]
        for t in range(T)
    ]
    btile = b_v[:]
    lane16 = lax.iota(jnp.int32, L) * L
    for m in range(L):
      idxb[pl.ds(m * L, L)] = lane16 + m
    lane = lax.iota(jnp.int32, L)
    row4 = lax.shift_right_logical(lane, 2)
    col4 = lane - row4 * T

    def issue(k_chunk, sr, dr, gs, gd):
      off = k_chunk * CH
      pltpu.async_copy(table_hbm.at[sidx.at[pl.ds(off, CH)]], sr, gs)
      pltpu.async_copy(table_hbm.at[didx.at[pl.ds(off, CH)]], dr, gd)

    def wait_gathers(sr, dr, gs, gd):
      pltpu.make_async_copy(table_hbm.at[sidx.at[pl.ds(0, CH)]], sr, gs).wait()
      pltpu.make_async_copy(table_hbm.at[didx.at[pl.ds(0, CH)]], dr, gd).wait()

    def compute(sr, dr, ob, ngrp):
      nhalf = max(ngrp // 2, 1)

      def half_loop(g0):
        @plsc.parallel_loop(g0, g0 + nhalf, unroll=1)
        def group(g):
          pb = pbufs.at[pl.ds((g - g0) * L * L, L * L)]
          for j in range(T):
            e = T * g + j
            h = []
            for v4 in range(DV // 2):
              sbf = plsc.bitcast(sr[e, L * v4:L * (v4 + 1)], jnp.bfloat16)
              dbf = plsc.bitcast(dr[e, L * v4:L * (v4 + 1)], jnp.bfloat16)
              h32 = jnp.maximum(sbf + dbf, 0.0)
              ha, hb = plsc.unpack(h32, format=plsc.PackFormat.INTERLEAVED)
              h.append(ha)
              h.append(hb)
            for t in range(T):
              ps = [h[v] * wvec[t][v] for v in range(DV)]
              while len(ps) > 1:
                ps = [ps[x] + ps[x + 1] for x in range(0, len(ps), 2)]
              m = j * T + t
              plsc.store_scatter(pb, [idxb[pl.ds(m * L, L)]], ps[0])
          ov = pb[0:L] + btile
          for r in range(1, L):
            ov = ov + pb[L * r:L * (r + 1)]
          plsc.store_scatter(ob, [row4 + g * T, col4], ov)

      half_loop(0)
      if ngrp > 1:
        half_loop(nhalf)

    def out_start(k_chunk, ob, sem):
      pltpu.async_copy(
          ob, out_hbm2.at[pl.ds(base + k_chunk * CH, CH)], sem)

    def out_wait(ob, sem):
      pltpu.make_async_copy(
          ob, out_hbm2.at[pl.ds(base, CH)], sem).wait()

    issue(0, srA, drA, gsA, gdA)
    issue(1, srB, drB, gsB, gdB)

    def pair_body(i, _):
      k0 = 2 * i
      wait_gathers(srA, drA, gsA, gdA)

      @pl.when(i > 0)
      def _wA():
        out_wait(outA, oA)

      compute(srA, drA, outA, CH // T)
      out_start(k0, outA, oA)

      @pl.when(i < npair - 1)
      def _iA():
        issue(k0 + 2, srA, drA, gsA, gdA)

      wait_gathers(srB, drB, gsB, gdB)

      @pl.when(i > 0)
      def _wB():
        out_wait(outB, oB)

      compute(srB, drB, outB, CH // T)
      out_start(k0 + 1, outB, oB)

      @pl.when(i < npair - 1)
      def _iB():
        issue(k0 + 3, srB, drB, gsB, gdB)

      return _

    lax.fori_loop(0, npair, pair_body, None)

    if tail:
      toff = nfull * CH
      pltpu.async_copy(
          table_hbm.at[sidx.at[pl.ds(toff, tail)]], srA.at[pl.ds(0, tail)],
          gsA)
      pltpu.async_copy(
          table_hbm.at[didx.at[pl.ds(toff, tail)]], drA.at[pl.ds(0, tail)],
          gdA)
      pltpu.make_async_copy(
          table_hbm.at[sidx.at[pl.ds(toff, tail)]], srA.at[pl.ds(0, tail)],
          gsA).wait()
      pltpu.make_async_copy(
          table_hbm.at[didx.at[pl.ds(toff, tail)]], drA.at[pl.ds(0, tail)],
          gdA).wait()
      out_wait(outA, oA)
      compute(srA, drA, outA, tail // T)
      out_wait(outB, oB)
      pltpu.sync_copy(
          outA.at[pl.ds(0, tail), :],
          out_hbm2.at[pl.ds(base + toff, tail)])
    else:
      out_wait(outA, oA)
      out_wait(outB, oB)

  return k


_kern1 = _make_kernel(0, EPW1)
_kern2 = _make_kernel(0, EPW2)


def kernel(encoded_graph, edge_index, W, b):
  ei = edge_index.astype(jnp.int32)
  table_bf = jnp.pad(
      jax.lax.bitcast_convert_type(
          encoded_graph.astype(jnp.bfloat16).reshape(N, D // 2, 2), jnp.int32),
      ((0, 0), (0, D // 2)))
  wr = W.T.reshape(T, DV // 2, L, 2).transpose(0, 1, 3, 2)
  binit = jnp.tile(b, L // T)
  src = ei[0]
  dst = ei[1]
  e1 = EPW1 * NW
  o1 = _kern1(table_bf, src[:e1], dst[:e1], wr, binit)
  o2 = _kern2(table_bf, src[e1:], dst[e1:], wr, binit)
  return jnp.concatenate([o1, o2], axis=0)
